# Initial kernel scaffold; baseline (speedup 1.0000x reference)
import functools

import jax
import jax.numpy as jnp
from jax import lax
from jax.experimental import pallas as pl
from jax.experimental.pallas import tpu as pltpu
from jax.experimental.pallas import tpu_sc as plsc

B = 16384
F = 26
D = 16
NUM_BLOCKS = 5
NC = 2
NS = 16
NW = NC * NS
RPW = B // NW
CR = 64
CE = F * CR
NCHUNK = RPW // CR
G = CR // 16

_THRESH = (100000, 300000, 500000, 700000)


def _fm_body(ids_hbm, vals_hbm, emb_hbm, fm_hbm, mask_hbm, bias_hbm,
             out_hbm, ids_v, vals_v, emb_v, fmv_v, mask_v, bias_v, out_v,
             sem_e, sem_f):
    wid = lax.axis_index("s") * NC + lax.axis_index("c")
    base_col = wid * RPW

    pltpu.sync_copy(mask_hbm, mask_v)
    pltpu.sync_copy(bias_hbm, bias_v)
    bias_vec = bias_v[...]
    zeros = jnp.zeros((16,), jnp.float32)

    def chunk_body(c, _):
        col0 = base_col + c * CR
        pltpu.sync_copy(ids_hbm.at[:, pl.ds(col0, CR)], ids_v)
        pltpu.sync_copy(vals_hbm.at[:, pl.ds(col0, CR)], vals_v)

        def fire(j, _):
            pltpu.make_async_copy(
                emb_hbm.at[ids_v.at[j]],
                emb_v.at[pl.ds(j * CR, CR)], sem_e).start()
            pltpu.make_async_copy(
                fm_hbm.at[ids_v.at[j]],
                fmv_v.at[pl.ds(j * CR, CR)], sem_f).start()
            return 0
        lax.fori_loop(0, F, fire, 0)

        def drain(j, _):
            pltpu.make_async_copy(
                emb_hbm.at[ids_v.at[0]],
                emb_v.at[pl.ds(0, CR)], sem_e).wait()
            pltpu.make_async_copy(
                fm_hbm.at[ids_v.at[0]],
                fmv_v.at[pl.ds(0, CR)], sem_f).wait()
            return 0
        lax.fori_loop(0, F, drain, 0)

        def group_body(g, _):
            lane = lax.iota(jnp.int32, 16)

            def field_body(f, carry):
                accs = carry[:D]
                accsq = carry[D]
                lin = carry[D + 1]
                id_vec = ids_v[f, pl.ds(g * 16, 16)]
                val_vec = vals_v[f, pl.ds(g * 16, 16)]
                fmvv = fmv_v[pl.ds(f * CR + g * 16, 16)]
                lin = lin + val_vec * fmvv
                mid = ((id_vec >= _THRESH[0]).astype(jnp.int32)
                       + (id_vec >= _THRESH[1]).astype(jnp.int32)
                       + (id_vec >= _THRESH[2]).astype(jnp.int32)
                       + (id_vec >= _THRESH[3]).astype(jnp.int32))
                elem_idx = f * CR + g * 16 + lane
                new_accs = []
                for d in range(D):
                    dsplat = jnp.full((16,), d, jnp.int32)
                    e_d = plsc.load_gather(emb_v, [elem_idx, dsplat])
                    m_d = plsc.load_gather(mask_v, [mid, dsplat])
                    t = e_d * m_d * val_vec
                    new_accs.append(accs[d] + t)
                    accsq = accsq + t * t
                return tuple(new_accs) + (accsq, lin)

            init = tuple(zeros for _ in range(D)) + (zeros, bias_vec)
            res = lax.fori_loop(0, F, field_body, init)
            sq = res[0] * res[0]
            for d in range(1, D):
                sq = sq + res[d] * res[d]
            x = res[D + 1] + 0.5 * (sq - res[D])
            out_v[pl.ds(c * CR + g * 16, 16)] = 1.0 / (1.0 + jnp.exp(-x))
            return 0
        lax.fori_loop(0, G, group_body, 0)
        return 0

    lax.fori_loop(0, NCHUNK, chunk_body, 0)
    pltpu.sync_copy(out_v, out_hbm.at[pl.ds(base_col, RPW)])


@jax.jit
def _fm_sc(ids_t, vals_t, emb_table, fm_flat, mask_table, bias_vec):
    mesh = plsc.VectorSubcoreMesh(core_axis_name="c", subcore_axis_name="s")
    return pl.kernel(
        _fm_body,
        out_type=jax.ShapeDtypeStruct((B,), jnp.float32),
        mesh=mesh,
        scratch_types=[
            pltpu.VMEM((F, CR), jnp.int32),
            pltpu.VMEM((F, CR), jnp.float32),
            pltpu.VMEM((CE, D), jnp.float32),
            pltpu.VMEM((CE,), jnp.float32),
            pltpu.VMEM((NUM_BLOCKS, 16), jnp.float32),
            pltpu.VMEM((16,), jnp.float32),
            pltpu.VMEM((RPW,), jnp.float32),
            pltpu.SemaphoreType.DMA,
            pltpu.SemaphoreType.DMA,
        ],
    )(ids_t, vals_t, emb_table, fm_flat, mask_table, bias_vec)


def kernel(feature_ids, feature_vals, emb_table, alpha, fm_w, fm_bias):
    ids_t = feature_ids.T.astype(jnp.int32)
    vals_t = feature_vals.T
    fm_flat = fm_w.reshape(-1)
    s = jnp.arange(NUM_BLOCKS, dtype=jnp.float32)
    abw = jnp.clip(alpha[None, :] - s[:, None], 0.0, 1.0)
    mask = jnp.repeat(abw, D // abw.shape[1], axis=1)
    msum = mask.sum(axis=1, keepdims=True)
    mask_n = mask / (msum + 1e-6) * 4.0
    bias_vec = jnp.full((16,), fm_bias, dtype=jnp.float32)
    return _fm_sc(ids_t, vals_t, emb_table, fm_flat, mask_n, bias_vec)


# speedup vs baseline: 3.7896x; 3.7896x over previous
import functools

import jax
import jax.numpy as jnp
from jax import lax
from jax.experimental import pallas as pl
from jax.experimental.pallas import tpu as pltpu
from jax.experimental.pallas import tpu_sc as plsc

B = 16384
F = 26
D = 16
NUM_BLOCKS = 5
NC = 2
NS = 16
NW = NC * NS
RPW = B // NW
CR = 128
CE = F * CR
NCHUNK = RPW // CR
G = CR // 16

_THRESH = (100000, 300000, 500000, 700000)

USE_FM = True


def _fm_body(ids_hbm, vals_hbm, emb_hbm, fm_hbm, mask_hbm, bias_hbm,
             out_hbm, ids_v, vals_v, emb_v, fmv_v, mask_v, bias_v, out_v,
             sem_e, sem_f):
    wid = lax.axis_index("s") * NC + lax.axis_index("c")
    base_col = wid * RPW

    pltpu.sync_copy(mask_hbm, mask_v)
    pltpu.sync_copy(bias_hbm, bias_v)
    bias_vec = bias_v[...]
    zeros = jnp.zeros((16,), jnp.float32)

    def chunk_body(c, _):
        col0 = pl.multiple_of(base_col + c * CR, CR)
        pltpu.sync_copy(ids_hbm.at[:, pl.ds(col0, CR)], ids_v)
        pltpu.sync_copy(vals_hbm.at[:, pl.ds(col0, CR)], vals_v)

        for j in range(F):
            pltpu.make_async_copy(
                emb_hbm.at[ids_v.at[j]],
                emb_v.at[pl.ds(j * CR, CR)], sem_e).start()
            if USE_FM:
                pltpu.make_async_copy(
                    fm_hbm.at[ids_v.at[j]],
                    fmv_v.at[pl.ds(j * CR, CR)], sem_f).start()
        for j in range(F):
            pltpu.make_async_copy(
                emb_hbm.at[ids_v.at[0]],
                emb_v.at[pl.ds(0, CR)], sem_e).wait()
            if USE_FM:
                pltpu.make_async_copy(
                    fm_hbm.at[ids_v.at[0]],
                    fmv_v.at[pl.ds(0, CR)], sem_f).wait()

        def group_body(g, _):
            lane = lax.iota(jnp.int32, 16)

            def field_body(f, carry):
                accs = carry[:D]
                accsq = carry[D]
                lin = carry[D + 1]
                id_vec = ids_v[f, pl.ds(g * 16, 16)]
                val_vec = vals_v[f, pl.ds(g * 16, 16)]
                fmvv = fmv_v[pl.ds(f * CR + g * 16, 16)]
                lin = lin + val_vec * fmvv
                mid = ((id_vec >= _THRESH[0]).astype(jnp.int32)
                       + (id_vec >= _THRESH[1]).astype(jnp.int32)
                       + (id_vec >= _THRESH[2]).astype(jnp.int32)
                       + (id_vec >= _THRESH[3]).astype(jnp.int32))
                elem_idx = f * CR + g * 16 + lane
                new_accs = []
                for d in range(D):
                    dsplat = jnp.full((16,), d, jnp.int32)
                    e_d = plsc.load_gather(emb_v, [elem_idx, dsplat])
                    m_d = plsc.load_gather(mask_v, [mid, dsplat])
                    t = e_d * m_d * val_vec
                    new_accs.append(accs[d] + t)
                    accsq = accsq + t * t
                return tuple(new_accs) + (accsq, lin)

            init = tuple(zeros for _ in range(D)) + (zeros, bias_vec)
            res = lax.fori_loop(0, F, field_body, init)
            sq = res[0] * res[0]
            for d in range(1, D):
                sq = sq + res[d] * res[d]
            x = res[D + 1] + 0.5 * (sq - res[D])
            out_v[pl.ds(c * CR + g * 16, 16)] = 1.0 / (1.0 + jnp.exp(-x))
            return 0
        lax.fori_loop(0, G, group_body, 0)
        return 0

    lax.fori_loop(0, NCHUNK, chunk_body, 0)
    pltpu.sync_copy(out_v, out_hbm.at[pl.ds(base_col, RPW)])


@jax.jit
def _fm_sc(ids_t, vals_t, emb_table, fm_flat, mask_table, bias_vec):
    mesh = plsc.VectorSubcoreMesh(core_axis_name="c", subcore_axis_name="s")
    return pl.kernel(
        _fm_body,
        out_type=jax.ShapeDtypeStruct((B,), jnp.float32),
        mesh=mesh,
        compiler_params=pltpu.CompilerParams(
            use_tc_tiling_on_sc=False, needs_layout_passes=False),
        scratch_types=[
            pltpu.VMEM((F, CR), jnp.int32),
            pltpu.VMEM((F, CR), jnp.float32),
            pltpu.VMEM((CE, D), jnp.float32),
            pltpu.VMEM((CE,), jnp.float32),
            pltpu.VMEM((NUM_BLOCKS, 16), jnp.float32),
            pltpu.VMEM((16,), jnp.float32),
            pltpu.VMEM((RPW,), jnp.float32),
            pltpu.SemaphoreType.DMA,
            pltpu.SemaphoreType.DMA,
        ],
    )(ids_t, vals_t, emb_table, fm_flat, mask_table, bias_vec)


def kernel(feature_ids, feature_vals, emb_table, alpha, fm_w, fm_bias):
    ids_t = feature_ids.T.astype(jnp.int32)
    vals_t = feature_vals.T
    fm_flat = fm_w.reshape(-1)
    s = jnp.arange(NUM_BLOCKS, dtype=jnp.float32)
    abw = jnp.clip(alpha[None, :] - s[:, None], 0.0, 1.0)
    mask = jnp.repeat(abw, D // abw.shape[1], axis=1)
    msum = mask.sum(axis=1, keepdims=True)
    mask_n = mask / (msum + 1e-6) * 4.0
    bias_vec = jnp.full((16,), fm_bias, dtype=jnp.float32)
    return _fm_sc(ids_t, vals_t, emb_table, fm_flat, mask_n, bias_vec)
